# Initial kernel scaffold; baseline (speedup 1.0000x reference)
import jax
import jax.numpy as jnp
from jax.experimental import pallas as pl


def kernel(x, edge_index, W1, b1, W2, b2, Wo, bo):
    raise NotImplementedError("write your pallas kernel here")


# speedup vs baseline: 26.7165x; 26.7165x over previous
import functools

import jax
import jax.numpy as jnp
from jax import lax
from jax.experimental import pallas as pl
from jax.experimental.pallas import tpu as pltpu
from jax.experimental.pallas import tpu_sc as plsc

_N = 10000
_E = 320000
_D = 128
_H = 16
_C = 40

_NC = 2
_NS = 16
_NW = _NC * _NS
_CH = 128
_RPW = 80
_ROWS = _RPW * _NW
_EP = _ROWS * _CH
_NACC = 10240
_ZR = _NACC // _NS

_mesh = plsc.VectorSubcoreMesh(core_axis_name="c", subcore_axis_name="s")


@functools.partial(
    pl.kernel,
    out_type=jax.ShapeDtypeStruct((_NC, _NACC), jnp.float32),
    mesh=_mesh,
    compiler_params=pltpu.CompilerParams(use_tc_tiling_on_sc=False),
    scratch_types=[
        pltpu.VMEM((_RPW, _CH), jnp.int32),
        pltpu.VMEM((_CH,), jnp.float32),
        pltpu.VMEM((_ZR,), jnp.float32),
        pltpu.VMEM_SHARED((_NACC,), jnp.float32),
    ],
)
def _deg_kernel(dst_hbm, out_hbm, dst_v, ones_v, zb_v, acc_sh):
    cid = lax.axis_index("c")
    sid = lax.axis_index("s")
    wid = sid * _NC + cid

    def zstep(i, _):
        zb_v[pl.ds(i * 16, 16)] = jnp.zeros((16,), jnp.float32)
        return 0

    lax.fori_loop(0, _ZR // 16, zstep, 0)
    for i in range(_CH // 16):
        ones_v[pl.ds(i * 16, 16)] = jnp.ones((16,), jnp.float32)
    pltpu.sync_copy(zb_v, acc_sh.at[pl.ds(sid * _ZR, _ZR)])
    plsc.subcore_barrier()

    pltpu.sync_copy(dst_hbm.at[pl.ds(wid * _RPW, _RPW)], dst_v)

    def step(j, _):
        pltpu.sync_copy(ones_v, acc_sh.at[dst_v.at[j]], add=True)
        return 0

    lax.fori_loop(0, _RPW, step, 0)
    plsc.subcore_barrier()
    pltpu.sync_copy(acc_sh.at[pl.ds(sid * _ZR, _ZR)],
                    out_hbm.at[cid, pl.ds(sid * _ZR, _ZR)])


@functools.partial(
    pl.kernel,
    out_type=jax.ShapeDtypeStruct((_NC, _NACC, _H), jnp.float32),
    mesh=_mesh,
    compiler_params=pltpu.CompilerParams(use_tc_tiling_on_sc=False),
    scratch_types=[
        pltpu.VMEM((_RPW, _CH), jnp.int32),
        pltpu.VMEM((_RPW, _CH), jnp.int32),
        pltpu.VMEM((_CH, _H), jnp.float32),
        pltpu.VMEM((_ZR, _H), jnp.float32),
        pltpu.VMEM_SHARED((_NACC, _H), jnp.float32),
        pltpu.SemaphoreType.DMA,
    ],
)
def _agg_kernel(hn_hbm, src_hbm, dst_hbm, out_hbm,
                src_v, dst_v, rows_v, zb_v, acc_sh, sem):
    cid = lax.axis_index("c")
    sid = lax.axis_index("s")
    wid = sid * _NC + cid

    def zstep(i, _):
        zb_v[i] = jnp.zeros((_H,), jnp.float32)
        return 0

    lax.fori_loop(0, _ZR, zstep, 0)
    pltpu.sync_copy(zb_v, acc_sh.at[pl.ds(sid * _ZR, _ZR)])
    plsc.subcore_barrier()

    pltpu.sync_copy(src_hbm.at[pl.ds(wid * _RPW, _RPW)], src_v)
    pltpu.sync_copy(dst_hbm.at[pl.ds(wid * _RPW, _RPW)], dst_v)

    def step(j, _):
        pltpu.async_copy(hn_hbm.at[src_v.at[j]], rows_v, sem).wait()
        pltpu.sync_copy(rows_v, acc_sh.at[dst_v.at[j]], add=True)
        return 0

    lax.fori_loop(0, _RPW, step, 0)
    plsc.subcore_barrier()
    pltpu.sync_copy(acc_sh.at[pl.ds(sid * _ZR, _ZR)],
                    out_hbm.at[cid, pl.ds(sid * _ZR, _ZR)])


def _dense1_body(d0, d1, x, w1, hn, dv):
    dinv = lax.rsqrt(d0[...] + d1[...] + 1.0)
    dv[...] = dinv
    hn[...] = jnp.dot(x[...], w1[...], preferred_element_type=jnp.float32) * dinv


def _dense2_body(p0, p1, hn1, dv, b1, w2, hn2):
    s = jnp.maximum(dv[...] * (p0[...] + p1[...] + hn1[...]) + b1[...], 0.0)
    hn2[...] = jnp.dot(s, w2[...], preferred_element_type=jnp.float32) * dv[...]


def _dense3_body(p0, p1, hn2, dv, b2, wo, bo, out):
    s = jnp.maximum(dv[...] * (p0[...] + p1[...] + hn2[...]) + b2[...], 0.0)
    logits = jnp.dot(s, wo[...], preferred_element_type=jnp.float32) + bo[...]
    m = jnp.max(logits, axis=1, keepdims=True)
    lse = jnp.log(jnp.sum(jnp.exp(logits - m), axis=1, keepdims=True)) + m
    out[...] = logits - lse


def kernel(x, edge_index, W1, b1, W2, b2, Wo, bo):
    src = edge_index[0]
    dst = edge_index[1]
    pad = _EP - _E
    srcp = jnp.concatenate([src, jnp.zeros((pad,), jnp.int32)]).reshape(_ROWS, _CH)
    dstp = jnp.concatenate([dst, jnp.full((pad,), _N, jnp.int32)]).reshape(_ROWS, _CH)

    degp = _deg_kernel(dstp)
    d0 = degp[0, :_N].reshape(_N, 1)
    d1 = degp[1, :_N].reshape(_N, 1)

    hn1, dinv = pl.pallas_call(
        _dense1_body,
        out_shape=[jax.ShapeDtypeStruct((_N, _H), jnp.float32),
                   jax.ShapeDtypeStruct((_N, 1), jnp.float32)],
    )(d0, d1, x, W1)

    a1 = _agg_kernel(hn1, srcp, dstp)
    hn2 = pl.pallas_call(
        _dense2_body,
        out_shape=jax.ShapeDtypeStruct((_N, _H), jnp.float32),
    )(a1[0, :_N], a1[1, :_N], hn1, dinv, b1.reshape(1, _H), W2)

    a2 = _agg_kernel(hn2, srcp, dstp)
    out = pl.pallas_call(
        _dense3_body,
        out_shape=jax.ShapeDtypeStruct((_N, _C), jnp.float32),
    )(a2[0, :_N], a2[1, :_N], hn2, dinv, b2.reshape(1, _H), Wo, bo.reshape(1, _C))
    return out


# speedup vs baseline: 33.6797x; 1.2606x over previous
import functools

import jax
import jax.numpy as jnp
from jax import lax
from jax.experimental import pallas as pl
from jax.experimental.pallas import tpu as pltpu
from jax.experimental.pallas import tpu_sc as plsc

_N = 10000
_E = 320000
_D = 128
_H = 16
_C = 40

_NC = 2
_NS = 16
_NW = _NC * _NS
_CH = 128
_RPW = 80
_ROWS = _RPW * _NW
_EP = _ROWS * _CH
_NACC = 10240
_ZR = _NACC // _NS

_mesh = plsc.VectorSubcoreMesh(core_axis_name="c", subcore_axis_name="s")


@functools.partial(
    pl.kernel,
    out_type=jax.ShapeDtypeStruct((_NC, _NACC), jnp.float32),
    mesh=_mesh,
    compiler_params=pltpu.CompilerParams(use_tc_tiling_on_sc=False),
    scratch_types=[
        pltpu.VMEM((_RPW, _CH), jnp.int32),
        pltpu.VMEM((_CH,), jnp.float32),
        pltpu.VMEM((_ZR,), jnp.float32),
        pltpu.VMEM_SHARED((_NACC,), jnp.float32),
    ],
)
def _deg_kernel(dst_hbm, out_hbm, dst_v, ones_v, zb_v, acc_sh):
    cid = lax.axis_index("c")
    sid = lax.axis_index("s")
    wid = sid * _NC + cid

    def zstep(i, _):
        zb_v[pl.ds(i * 16, 16)] = jnp.zeros((16,), jnp.float32)
        return 0

    lax.fori_loop(0, _ZR // 16, zstep, 0)
    for i in range(_CH // 16):
        ones_v[pl.ds(i * 16, 16)] = jnp.ones((16,), jnp.float32)
    pltpu.sync_copy(zb_v, acc_sh.at[pl.ds(sid * _ZR, _ZR)])
    plsc.subcore_barrier()

    pltpu.sync_copy(dst_hbm.at[pl.ds(wid * _RPW, _RPW)], dst_v)

    def step(j, _):
        pltpu.sync_copy(ones_v, acc_sh.at[dst_v.at[j]], add=True)
        return 0

    lax.fori_loop(0, _RPW, step, 0)
    plsc.subcore_barrier()
    pltpu.sync_copy(acc_sh.at[pl.ds(sid * _ZR, _ZR)],
                    out_hbm.at[cid, pl.ds(sid * _ZR, _ZR)])


@functools.partial(
    pl.kernel,
    out_type=jax.ShapeDtypeStruct((_NC, _NACC, _H), jnp.float32),
    mesh=_mesh,
    compiler_params=pltpu.CompilerParams(use_tc_tiling_on_sc=False),
    scratch_types=[
        pltpu.VMEM((_RPW, _CH), jnp.int32),
        pltpu.VMEM((_RPW, _CH), jnp.int32),
        pltpu.VMEM((2, _CH, _H), jnp.float32),
        pltpu.VMEM((_ZR, _H), jnp.float32),
        pltpu.VMEM_SHARED((_NACC, _H), jnp.float32),
        pltpu.SemaphoreType.DMA,
    ],
)
def _agg_kernel(hn_hbm, src_hbm, dst_hbm, out_hbm,
                src_v, dst_v, rows_v, zb_v, acc_sh, sem):
    cid = lax.axis_index("c")
    sid = lax.axis_index("s")
    wid = sid * _NC + cid

    def zstep(i, _):
        zb_v[i] = jnp.zeros((_H,), jnp.float32)
        return 0

    lax.fori_loop(0, _ZR, zstep, 0)
    pltpu.sync_copy(zb_v, acc_sh.at[pl.ds(sid * _ZR, _ZR)])
    plsc.subcore_barrier()

    pltpu.sync_copy(src_hbm.at[pl.ds(wid * _RPW, _RPW)], src_v)
    pltpu.sync_copy(dst_hbm.at[pl.ds(wid * _RPW, _RPW)], dst_v)

    pltpu.async_copy(hn_hbm.at[src_v.at[0]], rows_v.at[0], sem)

    def step(j, _):
        nxt = j + 1

        @pl.when(nxt < _RPW)
        def _():
            pltpu.async_copy(hn_hbm.at[src_v.at[nxt]], rows_v.at[nxt % 2], sem)

        pltpu.make_async_copy(hn_hbm.at[src_v.at[j]], rows_v.at[j % 2], sem).wait()
        pltpu.sync_copy(rows_v.at[j % 2], acc_sh.at[dst_v.at[j]], add=True)
        return 0

    lax.fori_loop(0, _RPW, step, 0)
    plsc.subcore_barrier()
    pltpu.sync_copy(acc_sh.at[pl.ds(sid * _ZR, _ZR)],
                    out_hbm.at[cid, pl.ds(sid * _ZR, _ZR)])


def _dense1_body(d0, d1, x, w1, hn, dv):
    dinv = lax.rsqrt(d0[...] + d1[...] + 1.0)
    dv[...] = dinv
    hn[...] = jnp.dot(x[...], w1[...], preferred_element_type=jnp.float32) * dinv


def _dense2_body(p0, p1, hn1, dv, b1, w2, hn2):
    s = jnp.maximum(dv[...] * (p0[...] + p1[...] + hn1[...]) + b1[...], 0.0)
    hn2[...] = jnp.dot(s, w2[...], preferred_element_type=jnp.float32) * dv[...]


def _dense3_body(p0, p1, hn2, dv, b2, wo, bo, out):
    s = jnp.maximum(dv[...] * (p0[...] + p1[...] + hn2[...]) + b2[...], 0.0)
    logits = jnp.dot(s, wo[...], preferred_element_type=jnp.float32) + bo[...]
    m = jnp.max(logits, axis=1, keepdims=True)
    lse = jnp.log(jnp.sum(jnp.exp(logits - m), axis=1, keepdims=True)) + m
    out[...] = logits - lse


def kernel(x, edge_index, W1, b1, W2, b2, Wo, bo):
    src = edge_index[0]
    dst = edge_index[1]
    pad = _EP - _E
    srcp = jnp.concatenate([src, jnp.zeros((pad,), jnp.int32)]).reshape(_ROWS, _CH)
    dstp = jnp.concatenate([dst, jnp.full((pad,), _N, jnp.int32)]).reshape(_ROWS, _CH)

    degp = _deg_kernel(dstp)
    d0 = degp[0, :_N].reshape(_N, 1)
    d1 = degp[1, :_N].reshape(_N, 1)

    hn1, dinv = pl.pallas_call(
        _dense1_body,
        out_shape=[jax.ShapeDtypeStruct((_N, _H), jnp.float32),
                   jax.ShapeDtypeStruct((_N, 1), jnp.float32)],
    )(d0, d1, x, W1)

    a1 = _agg_kernel(hn1, srcp, dstp)
    hn2 = pl.pallas_call(
        _dense2_body,
        out_shape=jax.ShapeDtypeStruct((_N, _H), jnp.float32),
    )(a1[0, :_N], a1[1, :_N], hn1, dinv, b1.reshape(1, _H), W2)

    a2 = _agg_kernel(hn2, srcp, dstp)
    out = pl.pallas_call(
        _dense3_body,
        out_shape=jax.ShapeDtypeStruct((_N, _C), jnp.float32),
    )(a2[0, :_N], a2[1, :_N], hn2, dinv, b2.reshape(1, _H), Wo, bo.reshape(1, _C))
    return out
